# baseline (speedup 1.0000x reference)
import functools

import jax
import jax.numpy as jnp
from jax import lax
from jax.experimental import pallas as pl
from jax.experimental.pallas import tpu as pltpu
from jax.experimental.pallas import tpu_sc as plsc

VOCAB = 1000000
EMB = 64

_info = plsc.get_sparse_core_info()
NC, NS, L = _info.num_cores, _info.num_subcores, _info.num_lanes
NW = NC * NS

B_TOTAL = 4096 * 200
PER_W = B_TOTAL // NW
IDX_ROWS = PER_W // 128
GATHERS_PER_CHUNK = 4
CHUNK = GATHERS_PER_CHUNK * 128
N_CHUNKS = PER_W // CHUNK


def _body(idx_hbm, tab_hbm, out_hbm, idx_v, rows_v, gsem):
    c = lax.axis_index("c")
    s = lax.axis_index("s")
    wid = s * NC + c
    base = wid * PER_W

    pltpu.sync_copy(idx_hbm.at[wid], idx_v)

    def chunk_body(ci, _):
        handles = []
        for j in range(GATHERS_PER_CHUNK):
            h = pltpu.async_copy(
                tab_hbm.at[idx_v.at[ci * GATHERS_PER_CHUNK + j]],
                rows_v.at[pl.ds(j * 128, 128)],
                gsem,
            )
            handles.append(h)
        for h in handles:
            h.wait()
        pltpu.sync_copy(rows_v, out_hbm.at[pl.ds(base + ci * CHUNK, CHUNK)])
        return 0

    lax.fori_loop(0, N_CHUNKS, chunk_body, 0)


@jax.jit
def _gather(idx, tvectors):
    mesh = plsc.VectorSubcoreMesh(core_axis_name="c", subcore_axis_name="s")
    f = pl.kernel(
        _body,
        out_type=jax.ShapeDtypeStruct((B_TOTAL, EMB), jnp.float32),
        mesh=mesh,
        scratch_types=[
            pltpu.VMEM((IDX_ROWS, 128), jnp.int32),
            pltpu.VMEM((CHUNK, EMB), jnp.float32),
            pltpu.SemaphoreType.DMA,
        ],
        compiler_params=pltpu.CompilerParams(use_tc_tiling_on_sc=False),
    )
    return f(idx, tvectors)


def kernel(data, tvectors):
    idx = data.astype(jnp.int32).reshape(NW, IDX_ROWS, 128)
    out = _gather(idx, tvectors)
    return out.reshape(data.shape[0], data.shape[1], EMB)


# speedup vs baseline: 1.0233x; 1.0233x over previous
import functools

import jax
import jax.numpy as jnp
from jax import lax
from jax.experimental import pallas as pl
from jax.experimental.pallas import tpu as pltpu
from jax.experimental.pallas import tpu_sc as plsc

VOCAB = 1000000
EMB = 64

_info = plsc.get_sparse_core_info()
NC, NS, L = _info.num_cores, _info.num_subcores, _info.num_lanes
NW = NC * NS

B_TOTAL = 4096 * 200
PER_W = B_TOTAL // NW
IDX_ROWS = PER_W // 128
GATHERS_PER_CHUNK = 2
CHUNK = GATHERS_PER_CHUNK * 128
N_CHUNKS = PER_W // CHUNK
NBUF = 4
LAG = 2


def _body(idx_hbm, tab_hbm, out_hbm, idx_v, rows, gsems, ssems):
    c = lax.axis_index("c")
    s = lax.axis_index("s")
    wid = s * NC + c
    base = wid * PER_W

    pltpu.sync_copy(idx_hbm.at[wid], idx_v)

    def fire_gathers(ci, b):
        for j in range(GATHERS_PER_CHUNK):
            pltpu.async_copy(
                tab_hbm.at[idx_v.at[ci * GATHERS_PER_CHUNK + j]],
                rows[b].at[pl.ds(j * 128, 128)],
                gsems[b],
            )

    def wait_gathers(b):
        pltpu.make_async_copy(
            tab_hbm.at[pl.ds(0, CHUNK)], rows[b], gsems[b]
        ).wait()

    def fire_store(ci, b):
        pltpu.async_copy(
            rows[b], out_hbm.at[pl.ds(base + ci * CHUNK, CHUNK)], ssems[b]
        )

    def wait_store(b):
        pltpu.make_async_copy(
            rows[b], out_hbm.at[pl.ds(base, CHUNK)], ssems[b]
        ).wait()

    def group(g, _):
        for b in range(NBUF):
            ci = g * NBUF + b
            @pl.when(g >= 1)
            def _():
                wait_store(b)
            fire_gathers(ci, b)
            bb = (b + NBUF - LAG) % NBUF
            if b >= LAG:
                wait_gathers(bb)
                fire_store(ci - LAG, bb)
            else:
                @pl.when(g >= 1)
                def _():
                    wait_gathers(bb)
                    fire_store(ci - LAG, bb)
        return 0

    lax.fori_loop(0, N_CHUNKS // NBUF, group, 0)

    for ci in range(N_CHUNKS - LAG, N_CHUNKS):
        b = ci % NBUF
        wait_gathers(b)
        fire_store(ci, b)
    for b in range(NBUF):
        wait_store(b)


@jax.jit
def _gather(idx, tvectors):
    mesh = plsc.VectorSubcoreMesh(core_axis_name="c", subcore_axis_name="s")
    f = pl.kernel(
        _body,
        out_type=jax.ShapeDtypeStruct((B_TOTAL, EMB), jnp.float32),
        mesh=mesh,
        scratch_types=[
            pltpu.VMEM((IDX_ROWS, 128), jnp.int32),
            [pltpu.VMEM((CHUNK, EMB), jnp.float32) for _ in range(NBUF)],
            [pltpu.SemaphoreType.DMA for _ in range(NBUF)],
            [pltpu.SemaphoreType.DMA for _ in range(NBUF)],
        ],
        compiler_params=pltpu.CompilerParams(use_tc_tiling_on_sc=False),
    )
    return f(idx, tvectors)


def kernel(data, tvectors):
    idx = data.astype(jnp.int32).reshape(NW, IDX_ROWS, 128)
    out = _gather(idx, tvectors)
    return out.reshape(data.shape[0], data.shape[1], EMB)


# speedup vs baseline: 1.0795x; 1.0549x over previous
import functools

import jax
import jax.numpy as jnp
from jax import lax
from jax.experimental import pallas as pl
from jax.experimental.pallas import tpu as pltpu
from jax.experimental.pallas import tpu_sc as plsc

VOCAB = 1000000
EMB = 64

_info = plsc.get_sparse_core_info()
NC, NS, L = _info.num_cores, _info.num_subcores, _info.num_lanes
NW = NC * NS

B_TOTAL = 4096 * 200
PER_W = B_TOTAL // NW
V_ROWS = PER_W // 128
CHUNK = 128
GPC = CHUNK // 128
N_CHUNKS = PER_W // CHUNK


def _body(idx_hbm, tab_hbm, out_hbm, v_all, idx2, par, g, o, gsems, ssems):
    c = lax.axis_index("c")
    s = lax.axis_index("s")
    wid = s * NC + c
    base = wid * PER_W

    pltpu.sync_copy(idx_hbm.at[wid], v_all)

    def compute_idx(ci, b):
        for j in range(GPC):
            row = ci * GPC + j
            for q in range(128 // L):
                v = v_all[row, pl.ds(q * L, L)]
                idx2[b][j, pl.ds(q * L, L)] = v >> 1
                par[b][pl.ds(j * 128 + q * L, L)] = (v & 1) * 64

    def fire_gathers(b):
        for j in range(GPC):
            pltpu.async_copy(
                tab_hbm.at[idx2[b].at[j]],
                g[b].at[pl.ds(j * 128, 128)],
                gsems[b],
            )

    def wait_gathers(b):
        pltpu.make_async_copy(
            tab_hbm.at[pl.ds(0, CHUNK)], g[b], gsems[b]
        ).wait()

    def select(b):
        def row_body(r16, _):
            pvec = par[b][pl.ds(r16 * L, L)]
            for u in range(L):
                r = r16 * L + u
                off = pvec[u]
                for cg in range(4):
                    o[b][r, pl.ds(cg * 16, 16)] = g[b][
                        r, pl.ds(off + cg * 16, 16)
                    ]
            return 0

        lax.fori_loop(0, CHUNK // L, row_body, 0)

    def fire_store(ci, b):
        pltpu.async_copy(
            o[b], out_hbm.at[pl.ds(base + ci * CHUNK, CHUNK)], ssems[b]
        )

    def wait_store(b):
        pltpu.make_async_copy(
            o[b], out_hbm.at[pl.ds(base, CHUNK)], ssems[b]
        ).wait()

    compute_idx(0, 0)
    fire_gathers(0)

    def group(gidx, _):
        for b in range(2):
            ci = gidx * 2 + b
            nb = 1 - b
            @pl.when(ci + 1 < N_CHUNKS)
            def _():
                compute_idx(ci + 1, nb)
                fire_gathers(nb)
            wait_gathers(b)
            @pl.when(ci >= 2)
            def _():
                wait_store(b)
            select(b)
            fire_store(ci, b)
        return 0

    lax.fori_loop(0, N_CHUNKS // 2, group, 0)
    for b in range(2):
        wait_store(b)


@jax.jit
def _gather(idx, tab2):
    mesh = plsc.VectorSubcoreMesh(core_axis_name="c", subcore_axis_name="s")
    f = pl.kernel(
        _body,
        out_type=jax.ShapeDtypeStruct((B_TOTAL, EMB), jnp.float32),
        mesh=mesh,
        scratch_types=[
            pltpu.VMEM((V_ROWS, 128), jnp.int32),
            [pltpu.VMEM((GPC, 128), jnp.int32) for _ in range(2)],
            [pltpu.VMEM((CHUNK,), jnp.int32) for _ in range(2)],
            [pltpu.VMEM((CHUNK, 128), jnp.float32) for _ in range(2)],
            [pltpu.VMEM((CHUNK, EMB), jnp.float32) for _ in range(2)],
            [pltpu.SemaphoreType.DMA for _ in range(2)],
            [pltpu.SemaphoreType.DMA for _ in range(2)],
        ],
        compiler_params=pltpu.CompilerParams(use_tc_tiling_on_sc=True),
    )
    return f(idx, tab2)


def kernel(data, tvectors):
    idx = data.astype(jnp.int32).reshape(NW, V_ROWS, 128)
    tab2 = tvectors.reshape(VOCAB // 2, 2 * EMB)
    out = _gather(idx, tab2)
    return out.reshape(data.shape[0], data.shape[1], EMB)


# speedup vs baseline: 1.1902x; 1.1025x over previous
import functools

import jax
import jax.numpy as jnp
from jax import lax
from jax.experimental import pallas as pl
from jax.experimental.pallas import tpu as pltpu
from jax.experimental.pallas import tpu_sc as plsc

VOCAB = 1000000
EMB = 64

_info = plsc.get_sparse_core_info()
NC, NS, L = _info.num_cores, _info.num_subcores, _info.num_lanes
NW = NC * NS

B_TOTAL = 4096 * 200
PER_W = B_TOTAL // NW
V_ROWS = PER_W // 128
CHUNK = 128
N_CHUNKS = PER_W // CHUNK
NBUF = 2


def _body(idx_hbm, tab_hbm, out_hbm, idx_v, g, o, gsems, ssems):
    c = lax.axis_index("c")
    s = lax.axis_index("s")
    wid = s * NC + c
    base = wid * PER_W

    pltpu.sync_copy(idx_hbm.at[wid], idx_v)

    def fire_gather(ci, b):
        pltpu.async_copy(tab_hbm.at[idx_v.at[ci]], g[b], gsems[b])

    def wait_gather(b):
        pltpu.make_async_copy(tab_hbm.at[pl.ds(0, CHUNK)], g[b], gsems[b]).wait()

    def compact(b):
        def row_body(r16, _):
            for u in range(L):
                r = r16 * L + u
                for cg in range(EMB // L):
                    o[b][r, pl.ds(cg * L, L)] = g[b][r, pl.ds(cg * L, L)]
            return 0

        lax.fori_loop(0, CHUNK // L, row_body, 0)

    def fire_store(ci, b):
        pltpu.async_copy(
            o[b], out_hbm.at[pl.ds(base + ci * CHUNK, CHUNK)], ssems[b]
        )

    def wait_store(b):
        pltpu.make_async_copy(
            o[b], out_hbm.at[pl.ds(base, CHUNK)], ssems[b]
        ).wait()

    fire_gather(0, 0)

    def group(gidx, _):
        for b in range(NBUF):
            ci = gidx * NBUF + b
            wait_gather(b)
            nb = 1 - b
            @pl.when(ci + 1 < N_CHUNKS)
            def _():
                fire_gather(ci + 1, nb)
            @pl.when(ci >= 2)
            def _():
                wait_store(b)
            compact(b)
            fire_store(ci, b)
        return 0

    lax.fori_loop(0, N_CHUNKS // NBUF, group, 0)
    for b in range(NBUF):
        wait_store(b)


@jax.jit
def _gather(idx, tab128):
    mesh = plsc.VectorSubcoreMesh(core_axis_name="c", subcore_axis_name="s")
    f = pl.kernel(
        _body,
        out_type=jax.ShapeDtypeStruct((B_TOTAL, EMB), jnp.float32),
        mesh=mesh,
        scratch_types=[
            pltpu.VMEM((V_ROWS, 128), jnp.int32),
            [pltpu.VMEM((CHUNK, 128), jnp.float32) for _ in range(NBUF)],
            [pltpu.VMEM((CHUNK, EMB), jnp.float32) for _ in range(NBUF)],
            [pltpu.SemaphoreType.DMA for _ in range(NBUF)],
            [pltpu.SemaphoreType.DMA for _ in range(NBUF)],
        ],
        compiler_params=pltpu.CompilerParams(use_tc_tiling_on_sc=True),
    )
    return f(idx, tab128)


def kernel(data, tvectors):
    idx = data.astype(jnp.int32).reshape(NW, V_ROWS, 128)
    tab128 = jnp.pad(tvectors, ((0, 0), (0, EMB)))
    out = _gather(idx, tab128)
    return out.reshape(data.shape[0], data.shape[1], EMB)
